# Initial kernel scaffold; baseline (speedup 1.0000x reference)
import jax
import jax.numpy as jnp
from jax.experimental import pallas as pl


def kernel(x, table):
    raise NotImplementedError("write your pallas kernel here")


# speedup vs baseline: 4.9476x; 4.9476x over previous
import functools

import jax
import jax.numpy as jnp
from jax import lax
from jax.experimental import pallas as pl
from jax.experimental.pallas import tpu as pltpu
from jax.experimental.pallas import tpu_sc as plsc

_NC = 2
_NS = 16
_NW = _NC * _NS


@functools.lru_cache(maxsize=None)
def _gather_call(B, E, CH):
    per_w = B // _NW
    n_chunks = per_w // CH
    mesh = plsc.VectorSubcoreMesh(core_axis_name="c", subcore_axis_name="s")

    @functools.partial(
        pl.kernel,
        mesh=mesh,
        out_type=jax.ShapeDtypeStruct((B, E), jnp.float32),
        scratch_types=[
            pltpu.VMEM((CH,), jnp.int32),
            pltpu.VMEM((CH, E), jnp.float32),
            pltpu.SemaphoreType.DMA,
        ],
        compiler_params=pltpu.CompilerParams(use_tc_tiling_on_sc=False),
    )
    def k(idx_hbm, table_hbm, out_hbm, idx_v, rows_v, sem):
        wid = lax.axis_index("s") * _NC + lax.axis_index("c")
        base = wid * per_w

        def body(i, carry):
            off = base + i * CH
            pltpu.sync_copy(idx_hbm.at[pl.ds(off, CH)], idx_v)
            pltpu.async_copy(table_hbm.at[idx_v], rows_v, sem).wait()
            pltpu.sync_copy(rows_v, out_hbm.at[pl.ds(off, CH)])
            return carry

        lax.fori_loop(0, n_chunks, body, 0)

    return k


def kernel(x, table):
    B0, H = x.shape
    E = table.shape[1]
    flat = x.reshape(B0 * H)
    out = _gather_call(B0 * H, E, 2048)(flat, table)
    return out.reshape(B0, H, E)


# speedup vs baseline: 5.0499x; 1.0207x over previous
import functools

import jax
import jax.numpy as jnp
from jax import lax
from jax.experimental import pallas as pl
from jax.experimental.pallas import tpu as pltpu
from jax.experimental.pallas import tpu_sc as plsc

_NC = 2
_NS = 16
_NW = _NC * _NS


@functools.lru_cache(maxsize=None)
def _gather_call(B, E, CH):
    per_w = B // _NW
    n_chunks = per_w // CH
    assert n_chunks >= 4 and n_chunks % 2 == 0
    mesh = plsc.VectorSubcoreMesh(core_axis_name="c", subcore_axis_name="s")

    @functools.partial(
        pl.kernel,
        mesh=mesh,
        out_type=jax.ShapeDtypeStruct((B, E), jnp.float32),
        scratch_types=[
            pltpu.VMEM((CH,), jnp.int32),
            pltpu.VMEM((CH,), jnp.int32),
            pltpu.VMEM((CH, E), jnp.float32),
            pltpu.VMEM((CH, E), jnp.float32),
            pltpu.SemaphoreType.DMA,
            pltpu.SemaphoreType.DMA,
            pltpu.SemaphoreType.DMA,
            pltpu.SemaphoreType.DMA,
        ],
        compiler_params=pltpu.CompilerParams(use_tc_tiling_on_sc=False),
    )
    def k(idx_hbm, table_hbm, out_hbm, iv0, iv1, r0, r1, g0, g1, o0, o1):
        wid = lax.axis_index("s") * _NC + lax.axis_index("c")
        base = wid * per_w
        iv = (iv0, iv1)
        rows = (r0, r1)
        g = (g0, g1)
        o = (o0, o1)

        for b in range(2):
            pltpu.sync_copy(idx_hbm.at[pl.ds(base + b * CH, CH)], iv[b])
            pltpu.async_copy(table_hbm.at[iv[b]], rows[b], g[b])

        def body(j, carry):
            for b in range(2):
                i = 2 * j + b
                off = base + i * CH
                pltpu.make_async_copy(table_hbm.at[iv[b]], rows[b], g[b]).wait()
                out_cp = pltpu.make_async_copy(
                    rows[b], out_hbm.at[pl.ds(off, CH)], o[b])
                out_cp.start()

                @pl.when(i + 2 < n_chunks)
                def _():
                    pltpu.sync_copy(
                        idx_hbm.at[pl.ds(off + 2 * CH, CH)], iv[b])
                    out_cp.wait()
                    pltpu.async_copy(table_hbm.at[iv[b]], rows[b], g[b])

            return carry

        lax.fori_loop(0, n_chunks // 2, body, 0)

        for b in range(2):
            off = base + (n_chunks - 2 + b) * CH
            pltpu.make_async_copy(
                rows[b], out_hbm.at[pl.ds(off, CH)], o[b]).wait()

    return k


def kernel(x, table):
    B0, H = x.shape
    E = table.shape[1]
    flat = x.reshape(B0 * H)
    out = _gather_call(B0 * H, E, 1600)(flat, table)
    return out.reshape(B0, H, E)
